# baseline (speedup 1.0000x reference)
import functools

import jax
import jax.numpy as jnp
from jax import lax
from jax.experimental import pallas as pl
from jax.experimental.pallas import tpu as pltpu
from jax.experimental.pallas import tpu_sc as plsc

NC = 2
NS = 16
NW = NC * NS
LANES = 16
GRP = 128


def _make_sc_embed(N, L, TYPE_V, TOK_V, D):
    nodes_per_w = N // NW
    sc_nodes = 32
    n_steps = nodes_per_w // sc_nodes
    idx_per_step = sc_nodes * L
    grps_per_step = idx_per_step // GRP
    tok_grps_per_w = nodes_per_w * L // GRP
    type_grps_per_w = nodes_per_w // GRP

    mesh = plsc.VectorSubcoreMesh(
        core_axis_name="c", subcore_axis_name="s",
        num_cores=NC, num_subcores=NS)

    @functools.partial(
        pl.kernel,
        mesh=mesh,
        compiler_params=pltpu.CompilerParams(use_tc_tiling_on_sc=False),
        out_type=(
            jax.ShapeDtypeStruct((N, D), jnp.float32),
            jax.ShapeDtypeStruct((N, D), jnp.float32),
        ),
        scratch_types=(
            pltpu.VMEM((tok_grps_per_w, GRP), jnp.int32),
            pltpu.VMEM((type_grps_per_w, GRP), jnp.int32),
            pltpu.VMEM((GRP, D), jnp.float32),
            pltpu.VMEM((idx_per_step, D), jnp.float32),
            pltpu.VMEM((sc_nodes, D), jnp.float32),
            pltpu.SemaphoreType.DMA,
        ),
    )
    def sc_embed(tok_ids_hbm, type_ids_hbm, type_tab_hbm, tok_tab_hbm,
                 type_out_hbm, tok_out_hbm,
                 tok_idx_v, type_idx_v, type_rows_v, rows_v, stage_v, sem):
        wid = lax.axis_index("s") * NC + lax.axis_index("c")
        node_base = wid * nodes_per_w

        pltpu.sync_copy(tok_ids_hbm.at[pl.ds(wid * tok_grps_per_w,
                                             tok_grps_per_w)], tok_idx_v)
        pltpu.sync_copy(type_ids_hbm.at[pl.ds(wid * type_grps_per_w,
                                              type_grps_per_w)], type_idx_v)

        def type_body(g, carry):
            pltpu.async_copy(type_tab_hbm.at[type_idx_v.at[g]],
                             type_rows_v, sem).wait()
            pltpu.sync_copy(type_rows_v,
                            type_out_hbm.at[pl.ds(node_base + g * GRP, GRP)])
            return carry
        lax.fori_loop(0, type_grps_per_w, type_body, 0)

        def step_body(s, carry):
            copies = []
            for g in range(grps_per_step):
                copies.append(pltpu.async_copy(
                    tok_tab_hbm.at[tok_idx_v.at[s * grps_per_step + g]],
                    rows_v.at[pl.ds(g * GRP, GRP)], sem))
            for cp in copies:
                cp.wait()

            def node_body(i, ncarry):
                base = i * L
                for dv in range(D // LANES):
                    sl = pl.ds(dv * LANES, LANES)
                    acc = rows_v[base, sl]
                    for l in range(1, L):
                        acc = acc + rows_v[base + l, sl]
                    stage_v[i, sl] = acc
                return ncarry
            lax.fori_loop(0, sc_nodes, node_body, 0)

            pltpu.sync_copy(
                stage_v,
                tok_out_hbm.at[pl.ds(node_base + s * sc_nodes, sc_nodes)])
            return carry
        lax.fori_loop(0, n_steps, step_body, 0)

    return sc_embed


def _mlp_body(t_ref, k_ref, w1_ref, w2_ref, b_ref, o_ref):
    y = (jnp.dot(t_ref[...], w1_ref[...], preferred_element_type=jnp.float32)
         + jnp.dot(k_ref[...], w2_ref[...], preferred_element_type=jnp.float32)
         + b_ref[...])
    o_ref[...] = jnp.maximum(y, 0.0)


def _mlp(type_emb, tok_sum, w1, w2, b2d):
    N, D = type_emb.shape
    C = w1.shape[1]
    blk = 2048
    grid = (N // blk,)
    return pl.pallas_call(
        _mlp_body,
        grid=grid,
        in_specs=[
            pl.BlockSpec((blk, D), lambda i: (i, 0)),
            pl.BlockSpec((blk, D), lambda i: (i, 0)),
            pl.BlockSpec((D, C), lambda i: (0, 0)),
            pl.BlockSpec((D, C), lambda i: (0, 0)),
            pl.BlockSpec((1, C), lambda i: (0, 0)),
        ],
        out_specs=pl.BlockSpec((blk, C), lambda i: (i, 0)),
        out_shape=jax.ShapeDtypeStruct((N, C), jnp.float32),
    )(type_emb, tok_sum, w1, w2, b2d)


def kernel(node_type_index, node_sub_token_ids, type_table, token_table, W, b):
    N, L = node_sub_token_ids.shape
    TYPE_V, D = type_table.shape
    TOK_V = token_table.shape[0]
    C = W.shape[0]

    tok_ids = node_sub_token_ids.astype(jnp.int32).reshape(N * L // GRP, GRP)
    type_ids = node_type_index.astype(jnp.int32).reshape(N // GRP, GRP)

    sc_embed = _make_sc_embed(N, L, TYPE_V, TOK_V, D)
    type_emb, tok_sum = sc_embed(tok_ids, type_ids, type_table, token_table)

    wt = W.T
    out = _mlp(type_emb, tok_sum, wt[:D], wt[D:], b.reshape(1, C))

    ast_node_index = jnp.arange(N, dtype=jnp.int32)
    return (ast_node_index, out)


# speedup vs baseline: 1.0771x; 1.0771x over previous
import functools

import jax
import jax.numpy as jnp
from jax import lax
from jax.experimental import pallas as pl
from jax.experimental.pallas import tpu as pltpu
from jax.experimental.pallas import tpu_sc as plsc

NC = 2
NS = 16
NW = NC * NS
LANES = 16
GRP = 128
CPAD = 128


def _make_sc_embed(N, L, TYPE_V, TOK_V, D):
    nodes_per_w = N // NW
    sc_nodes = 32
    n_steps = nodes_per_w // sc_nodes
    idx_per_step = sc_nodes * L
    grps_per_step = idx_per_step // GRP
    tok_grps_per_w = nodes_per_w * L // GRP

    mesh = plsc.VectorSubcoreMesh(
        core_axis_name="c", subcore_axis_name="s",
        num_cores=NC, num_subcores=NS)

    @functools.partial(
        pl.kernel,
        mesh=mesh,
        out_type=jax.ShapeDtypeStruct((N, CPAD), jnp.float32),
        scratch_types=(
            pltpu.VMEM((tok_grps_per_w, GRP), jnp.int32),
            pltpu.VMEM((nodes_per_w,), jnp.int32),
            pltpu.VMEM((idx_per_step, CPAD), jnp.float32),
            pltpu.VMEM((sc_nodes, CPAD), jnp.float32),
            pltpu.SemaphoreType.DMA,
            pltpu.SemaphoreType.DMA,
        ),
    )
    def sc_embed(tok_ids_hbm, type_ids_hbm, type_tab_hbm, tok_tab_hbm,
                 out_hbm,
                 tok_idx_v, type_idx_v, rows_v, stage_v, sem, sem2):
        wid = lax.axis_index("s") * NC + lax.axis_index("c")
        node_base = wid * nodes_per_w

        pltpu.sync_copy(tok_ids_hbm.at[pl.ds(wid * tok_grps_per_w,
                                             tok_grps_per_w)], tok_idx_v)
        pltpu.sync_copy(type_ids_hbm.at[pl.ds(node_base, nodes_per_w)],
                        type_idx_v)

        def step_body(s, carry):
            type_cp = pltpu.async_copy(
                type_tab_hbm.at[type_idx_v.at[pl.ds(s * sc_nodes, sc_nodes)]],
                stage_v, sem2)
            copies = []
            for g in range(grps_per_step):
                copies.append(pltpu.async_copy(
                    tok_tab_hbm.at[tok_idx_v.at[s * grps_per_step + g]],
                    rows_v.at[pl.ds(g * GRP, GRP)], sem))
            type_cp.wait()
            for cp in copies:
                cp.wait()

            def node_body(i, ncarry):
                base = i * L
                for dv in range(D // LANES):
                    sl = pl.ds(dv * LANES, LANES)
                    acc = rows_v[base, sl]
                    for l in range(1, L):
                        acc = acc + rows_v[base + l, sl]
                    stage_v[i, pl.ds(D + dv * LANES, LANES)] = acc
                return ncarry
            lax.fori_loop(0, sc_nodes, node_body, 0)

            pltpu.sync_copy(
                stage_v,
                out_hbm.at[pl.ds(node_base + s * sc_nodes, sc_nodes)])
            return carry
        lax.fori_loop(0, n_steps, step_body, 0)

    return sc_embed


def _mlp_body(x_ref, w_ref, b_ref, o_ref):
    y = jnp.dot(x_ref[...], w_ref[...],
                preferred_element_type=jnp.float32) + b_ref[...]
    o_ref[...] = jnp.maximum(y, 0.0)


def _mlp(x, wt, b2d):
    N, C = x.shape
    blk = 2048
    return pl.pallas_call(
        _mlp_body,
        grid=(N // blk,),
        in_specs=[
            pl.BlockSpec((blk, C), lambda i: (i, 0)),
            pl.BlockSpec((C, C), lambda i: (0, 0)),
            pl.BlockSpec((1, C), lambda i: (0, 0)),
        ],
        out_specs=pl.BlockSpec((blk, C), lambda i: (i, 0)),
        out_shape=jax.ShapeDtypeStruct((N, C), jnp.float32),
    )(x, wt, b2d)


def kernel(node_type_index, node_sub_token_ids, type_table, token_table, W, b):
    N, L = node_sub_token_ids.shape
    TYPE_V, D = type_table.shape
    TOK_V = token_table.shape[0]
    C = W.shape[0]

    tok_ids = node_sub_token_ids.astype(jnp.int32).reshape(N * L // GRP, GRP)
    type_ids = node_type_index.astype(jnp.int32)
    tok_pad = jnp.pad(token_table, ((0, 0), (0, CPAD - D)))
    type_pad = jnp.pad(type_table, ((0, 0), (0, CPAD - D)))

    sc_embed = _make_sc_embed(N, L, TYPE_V, TOK_V, D)
    node_emb = sc_embed(tok_ids, type_ids, type_pad, tok_pad)

    out = _mlp(node_emb, W.T, b.reshape(1, C))

    ast_node_index = jnp.arange(N, dtype=jnp.int32)
    return (ast_node_index, out)


# speedup vs baseline: 1.6743x; 1.5545x over previous
import functools

import jax
import jax.numpy as jnp
from jax import lax
from jax.experimental import pallas as pl
from jax.experimental.pallas import tpu as pltpu
from jax.experimental.pallas import tpu_sc as plsc

NC = 2
NS = 16
NW = NC * NS
LANES = 16
GRP = 128
CPAD = 128


def _make_sc_embed(N, L, TYPE_V, TOK_V, D):
    nodes_per_w = N // NW
    sc_nodes = 32
    n_steps = nodes_per_w // sc_nodes
    idx_per_step = sc_nodes * L
    grps_per_step = idx_per_step // GRP
    tok_grps_per_w = nodes_per_w * L // GRP

    mesh = plsc.VectorSubcoreMesh(
        core_axis_name="c", subcore_axis_name="s",
        num_cores=NC, num_subcores=NS)

    @functools.partial(
        pl.kernel,
        mesh=mesh,
        out_type=jax.ShapeDtypeStruct((N, CPAD), jnp.float32),
        scratch_types=(
            pltpu.VMEM((tok_grps_per_w, GRP), jnp.int32),
            pltpu.VMEM((nodes_per_w,), jnp.int32),
            pltpu.VMEM((idx_per_step, CPAD), jnp.float32),
            pltpu.VMEM((sc_nodes, CPAD), jnp.float32),
            pltpu.SemaphoreType.DMA,
            pltpu.SemaphoreType.DMA,
        ),
    )
    def sc_embed(tok_ids_hbm, type_ids_hbm, type_tab_hbm, tok_tab_hbm,
                 out_hbm,
                 tok_idx_v, type_idx_v, rows_v, stage_v, sem, sem2):
        wid = lax.axis_index("s") * NC + lax.axis_index("c")
        node_base = wid * nodes_per_w

        pltpu.sync_copy(tok_ids_hbm.at[pl.ds(wid * tok_grps_per_w,
                                             tok_grps_per_w)], tok_idx_v)
        pltpu.sync_copy(type_ids_hbm.at[pl.ds(node_base, nodes_per_w)],
                        type_idx_v)

        def step_body(s, carry):
            type_cp = pltpu.async_copy(
                type_tab_hbm.at[type_idx_v.at[pl.ds(s * sc_nodes, sc_nodes)]],
                stage_v, sem2)
            copies = []
            for g in range(grps_per_step):
                copies.append(pltpu.async_copy(
                    tok_tab_hbm.at[tok_idx_v.at[s * grps_per_step + g]],
                    rows_v.at[pl.ds(g * GRP, GRP)], sem))
            type_cp.wait()
            for cp in copies:
                cp.wait()

            def node_body(i, ncarry):
                base = i * L
                for dv in range(D // LANES):
                    sl = pl.ds(dv * LANES, LANES)
                    acc = rows_v[base, sl]
                    for l in range(1, L):
                        acc = acc + rows_v[base + l, sl]
                    stage_v[i, pl.ds(D + dv * LANES, LANES)] = acc
                return ncarry
            lax.fori_loop(0, sc_nodes, node_body, 0)

            pltpu.sync_copy(
                stage_v,
                out_hbm.at[pl.ds(node_base + s * sc_nodes, sc_nodes)])
            return carry
        lax.fori_loop(0, n_steps, step_body, 0)

    return sc_embed


def _repack_body(t_ref, i_ref, o_ref):
    r = jax.lax.dot_general(t_ref[...], i_ref[...],
                            (((0,), (0,)), ((), ())),
                            preferred_element_type=jnp.float32)
    o_ref[:, : r.shape[1]] = r
    o_ref[:, r.shape[1]:] = jnp.zeros_like(o_ref[:, r.shape[1]:])


def _repack(table_t, blk=8192):
    D, V = table_t.shape
    eye = jnp.eye(D, dtype=jnp.float32)
    grid = (pl.cdiv(V, blk),)
    return pl.pallas_call(
        _repack_body,
        grid=grid,
        in_specs=[
            pl.BlockSpec((D, blk), lambda i: (0, i)),
            pl.BlockSpec((D, D), lambda i: (0, 0)),
        ],
        out_specs=pl.BlockSpec((blk, CPAD), lambda i: (i, 0)),
        out_shape=jax.ShapeDtypeStruct((V, CPAD), jnp.float32),
    )(table_t, eye)


def _mlp_body(x_ref, w_ref, b_ref, o_ref):
    y = jnp.dot(x_ref[...], w_ref[...],
                preferred_element_type=jnp.float32) + b_ref[...]
    o_ref[...] = jnp.maximum(y, 0.0)


def _mlp(x, wt, b2d):
    N, C = x.shape
    blk = 2048
    return pl.pallas_call(
        _mlp_body,
        grid=(N // blk,),
        in_specs=[
            pl.BlockSpec((blk, C), lambda i: (i, 0)),
            pl.BlockSpec((C, C), lambda i: (0, 0)),
            pl.BlockSpec((1, C), lambda i: (0, 0)),
        ],
        out_specs=pl.BlockSpec((blk, C), lambda i: (i, 0)),
        out_shape=jax.ShapeDtypeStruct((N, C), jnp.float32),
    )(x, wt, b2d)


def kernel(node_type_index, node_sub_token_ids, type_table, token_table, W, b):
    N, L = node_sub_token_ids.shape
    TYPE_V, D = type_table.shape
    TOK_V = token_table.shape[0]
    C = W.shape[0]

    tok_ids = node_sub_token_ids.astype(jnp.int32).reshape(N * L // GRP, GRP)
    type_ids = node_type_index.astype(jnp.int32)
    tok_pad = _repack(token_table.T)
    type_pad = jnp.pad(type_table, ((0, 0), (0, CPAD - D)))

    sc_embed = _make_sc_embed(N, L, TYPE_V, TOK_V, D)
    node_emb = sc_embed(tok_ids, type_ids, type_pad, tok_pad)

    out = _mlp(node_emb, W.T, b.reshape(1, C))

    ast_node_index = jnp.arange(N, dtype=jnp.int32)
    return (ast_node_index, out)
